# baseline (speedup 1.0000x reference)
import jax
import jax.numpy as jnp
from jax import lax
from jax.experimental import pallas as pl
from jax.experimental.pallas import tpu as pltpu
from jax.experimental.pallas import tpu_sc as plsc

_B, _H, _W, _C = 64, 128, 128, 4
_HW = _H * _W
_L = 16
_GPR = _W // _L
_BIG = 1 << 30

_NC, _NS = 2, 16
_NW = _NC * _NS
_SPT = _B // _NW

_mesh = plsc.VectorSubcoreMesh(core_axis_name="c", subcore_axis_name="s",
                               num_cores=_NC, num_subcores=_NS)


def _oracle_body(x_hbm, out1,
                 xb3a, xb1a, xb3b, xb1b, zbuf,
                 i3a, i1a, i3b, i1b,
                 s3a, s1a, s3b, s1b, so1):
    wid = lax.axis_index("s") * _NC + lax.axis_index("c")
    iota = lax.iota(jnp.int32, _L)
    zeros_v = jnp.zeros((_L,), jnp.float32)
    big_v = jnp.full((_L,), _BIG, jnp.int32)
    lane0 = iota == 0
    cvecs = [iota + 16 * j for j in range(_GPR)]

    b0 = wid * _SPT
    b1 = b0 + 1

    def write_idx(ref, b, ch):
        base = b * (_H * _C) + ch
        for g in range(_GPR):
            ref[pl.ds(16 * g, _L)] = base + 4 * (iota + 16 * g)

    write_idx(i3a, b0, 3)
    write_idx(i1a, b0, 1)
    write_idx(i3b, b1, 3)
    write_idx(i1b, b1, 1)
    c3a = pltpu.async_copy(x_hbm.at[i3a], xb3a, s3a)
    c1a = pltpu.async_copy(x_hbm.at[i1a], xb1a, s1a)
    c3b = pltpu.async_copy(x_hbm.at[i3b], xb3b, s3b)
    c1b = pltpu.async_copy(x_hbm.at[i1b], xb1b, s1b)

    def zero_body(r, carry):
        for j in range(_GPR):
            zbuf[r, pl.ds(16 * j, _L)] = zeros_v
        return carry

    lax.fori_loop(0, _H, zero_body, 0)

    def phase1(buf):
        def p1_body(r, acc):
            rb_v = jnp.broadcast_to(r * _W, (_L,))
            for j in range(_GPR):
                v = buf[r, pl.ds(16 * j, _L)]
                cand = jnp.where(v == 1.0, rb_v + cvecs[j], _BIG)
                acc = jnp.minimum(acc, cand)
            return acc

        def chunk_cond(c):
            k, mn = c
            return (k < _H // 8) & (mn >= _BIG)

        def chunk_body(c):
            k, mn = c
            acc = lax.fori_loop(8 * k, 8 * k + 8, p1_body, big_v)
            return k + 1, jnp.minimum(mn, jnp.min(acc))

        _, mn = lax.while_loop(chunk_cond, chunk_body,
                               (jnp.int32(0), jnp.int32(_BIG)))
        return mn

    def phase2(buf, opp_min):
        has_opp = opp_min < _BIG
        opp_flat = jnp.where(has_opp, opp_min, 0)
        opp_r = opp_flat >> 7
        opp_c = opp_flat & (_W - 1)
        oc_v = jnp.broadcast_to(opp_c, (_L,))
        keys_j = []
        for j in range(_GPR):
            dc = cvecs[j] - oc_v
            keys_j.append(dc * dc * _HW + cvecs[j])

        def scan_row(row, s_row, acc):
            s_v = jnp.broadcast_to(s_row, (_L,))
            for j in range(_GPR):
                v = buf[row, pl.ds(16 * j, _L)]
                acc = jnp.minimum(acc,
                                  jnp.where(v == 1.0, s_v + keys_j[j], _BIG))
            return acc

        def out_cond(c):
            d, best = c
            return (d < _H) & (d * d * _HW <= best)

        def out_body(c):
            d, best = c
            dd = d * d * _HW
            r_lo = opp_r - d
            r_hi = opp_r + d
            s_lo = jnp.where(r_lo >= 0, dd + r_lo * _W, _BIG)
            s_hi = jnp.where(r_hi < _H, dd + r_hi * _W, _BIG)
            acc = scan_row(jnp.maximum(r_lo, 0), s_lo, big_v)
            acc = scan_row(jnp.minimum(r_hi, _H - 1), s_hi, acc)
            return d + 1, jnp.minimum(best, jnp.min(acc))

        _, fkey = lax.while_loop(out_cond, out_body,
                                 (jnp.int32(0), jnp.int32(_BIG)))

        tgt = jnp.where(fkey < _BIG, fkey & (_HW - 1), 0)
        opp_is_start = has_opp & (opp_flat == 3 * _W + 6)
        common = has_opp & (~opp_is_start)

        def count_food(_):
            def body(r, cnt):
                for j in range(_GPR):
                    m = buf[r, pl.ds(16 * j, _L)] == 1.0
                    cnt = cnt + m.astype(jnp.int32)
                return cnt

            return jnp.sum(lax.fori_loop(0, _H, body,
                                         jnp.zeros((_L,), jnp.int32)))

        n_food = lax.cond(common, lambda _: jnp.int32(2), count_food, 0)

        has_food = fkey < _BIG
        use_argmin = jnp.where(common, has_food, n_food == 1)
        target = jnp.where(use_argmin, tgt, 0)
        val = jnp.where(jnp.where(common, has_food, n_food > 0),
                        jnp.float32(1.0), jnp.float32(0.0))
        return target, val

    def scatter(target, val):
        tr_v = jnp.broadcast_to(target >> 7, (_L,))
        tc_v = jnp.broadcast_to(target & (_W - 1), (_L,))
        plsc.store_scatter(zbuf, [tr_v, tc_v],
                           jnp.broadcast_to(val, (_L,)), mask=lane0)
        return tr_v, tc_v

    c3a.wait()
    opp0 = phase1(xb3a)
    c1a.wait()
    target0, val0 = phase2(xb1a, opp0)
    tr0, tc0 = scatter(target0, val0)
    o1a = pltpu.async_copy(zbuf, out1.at[b0], so1)

    c3b.wait()
    opp1 = phase1(xb3b)
    c1b.wait()
    target1, val1 = phase2(xb1b, opp1)
    o1a.wait()
    plsc.store_scatter(zbuf, [tr0, tc0], zeros_v, mask=lane0)
    scatter(target1, val1)
    pltpu.sync_copy(zbuf, out1.at[b1])


_oracle = pl.kernel(
    _oracle_body,
    out_type=jax.ShapeDtypeStruct((_B, _H, _W), jnp.float32),
    mesh=_mesh,
    scratch_types=[pltpu.VMEM((_H, _W), jnp.float32),
                   pltpu.VMEM((_H, _W), jnp.float32),
                   pltpu.VMEM((_H, _W), jnp.float32),
                   pltpu.VMEM((_H, _W), jnp.float32),
                   pltpu.VMEM((_H, _W), jnp.float32),
                   pltpu.VMEM((_H,), jnp.int32),
                   pltpu.VMEM((_H,), jnp.int32),
                   pltpu.VMEM((_H,), jnp.int32),
                   pltpu.VMEM((_H,), jnp.int32),
                   pltpu.SemaphoreType.DMA,
                   pltpu.SemaphoreType.DMA,
                   pltpu.SemaphoreType.DMA,
                   pltpu.SemaphoreType.DMA,
                   pltpu.SemaphoreType.DMA],
    compiler_params=pltpu.CompilerParams(needs_layout_passes=False),
)


@jax.jit
def kernel(x, history):
    del history
    x_t = jnp.transpose(x, (0, 1, 3, 2)).reshape(_B * _H * _C, _W)
    out1 = _oracle(x_t)
    return out1, jnp.zeros((_B, _H, _W), jnp.float32)


# speedup vs baseline: 1.0611x; 1.0611x over previous
import jax
import jax.numpy as jnp
from jax import lax
from jax.experimental import pallas as pl
from jax.experimental.pallas import tpu as pltpu
from jax.experimental.pallas import tpu_sc as plsc

_B, _H, _W, _C = 64, 128, 128, 4
_HW = _H * _W
_L = 16
_GPR = _W // _L
_BIG = 1 << 30
_W3, _W1 = 16, 32

_NC, _NS = 2, 16
_NW = _NC * _NS
_SPT = _B // _NW

_mesh = plsc.VectorSubcoreMesh(core_axis_name="c", subcore_axis_name="s",
                               num_cores=_NC, num_subcores=_NS)


def _oracle_body(x_hbm, out1,
                 xb3a, xb1a, xb3b, xb1b, zbuf,
                 iw3a, iw1a, iw3b, iw1b, ifull,
                 s3a, s1a, s3b, s1b, so1):
    wid = lax.axis_index("s") * _NC + lax.axis_index("c")
    iota = lax.iota(jnp.int32, _L)
    zeros_v = jnp.zeros((_L,), jnp.float32)
    big_v = jnp.full((_L,), _BIG, jnp.int32)
    lane0 = iota == 0
    cvecs = [iota + 16 * j for j in range(_GPR)]

    b0 = wid * _SPT
    b1 = b0 + 1

    def write_idx(ref, b, ch, nrows):
        base = b * (_H * _C) + ch
        for g in range(nrows // _L):
            ref[pl.ds(_L * g, _L)] = base + 4 * (iota + _L * g)

    write_idx(iw3a, b0, 3, _W3)
    write_idx(iw1a, b0, 1, _W1)
    write_idx(iw3b, b1, 3, _W3)
    write_idx(iw1b, b1, 1, _W1)
    c3a = pltpu.async_copy(x_hbm.at[iw3a], xb3a.at[pl.ds(0, _W3)], s3a)
    c1a = pltpu.async_copy(x_hbm.at[iw1a], xb1a.at[pl.ds(0, _W1)], s1a)
    c3b = pltpu.async_copy(x_hbm.at[iw3b], xb3b.at[pl.ds(0, _W3)], s3b)
    c1b = pltpu.async_copy(x_hbm.at[iw1b], xb1b.at[pl.ds(0, _W1)], s1b)

    def zero_body(r, carry):
        for j in range(_GPR):
            zbuf[r, pl.ds(16 * j, _L)] = zeros_v
        return carry

    lax.fori_loop(0, _H, zero_body, 0)

    def phase1(buf, nchunks):
        def p1_body(r, acc):
            rb_v = jnp.broadcast_to(r * _W, (_L,))
            for j in range(_GPR):
                v = buf[r, pl.ds(16 * j, _L)]
                cand = jnp.where(v == 1.0, rb_v + cvecs[j], _BIG)
                acc = jnp.minimum(acc, cand)
            return acc

        def chunk_cond(c):
            k, mn = c
            return (k < nchunks) & (mn >= _BIG)

        def chunk_body(c):
            k, mn = c
            acc = lax.fori_loop(8 * k, 8 * k + 8, p1_body, big_v)
            return k + 1, jnp.minimum(mn, jnp.min(acc))

        _, mn = lax.while_loop(chunk_cond, chunk_body,
                               (jnp.int32(0), jnp.int32(_BIG)))
        return mn

    def phase2_scan(buf, opp_min, windowed):
        has_opp = opp_min < _BIG
        opp_flat = jnp.where(has_opp, opp_min, 0)
        opp_r = opp_flat >> 7
        opp_c = opp_flat & (_W - 1)
        oc_v = jnp.broadcast_to(opp_c, (_L,))
        keys_j = []
        for j in range(_GPR):
            dc = cvecs[j] - oc_v
            keys_j.append(dc * dc * _HW + cvecs[j])

        def scan_row(row, s_row, acc):
            s_v = jnp.broadcast_to(s_row, (_L,))
            for j in range(_GPR):
                v = buf[row, pl.ds(16 * j, _L)]
                acc = jnp.minimum(acc,
                                  jnp.where(v == 1.0, s_v + keys_j[j], _BIG))
            return acc

        def out_cond(c):
            d, best, bail = c
            return (d < _H) & (d * d * _HW <= best) & (~bail)

        def out_body(c):
            d, best, bail = c
            dd = d * d * _HW
            r_lo = opp_r - d
            r_hi = opp_r + d
            if windowed:
                bail = bail | ((r_hi < _H) & (r_hi >= _W1))
            s_lo = jnp.where(r_lo >= 0, dd + r_lo * _W, _BIG)
            s_hi = jnp.where(r_hi < _H, dd + r_hi * _W, _BIG)
            acc = scan_row(jnp.maximum(r_lo, 0), s_lo, big_v)
            acc = scan_row(jnp.minimum(r_hi, _H - 1), s_hi, acc)
            return d + 1, jnp.minimum(best, jnp.min(acc)), bail

        _, fkey, bail = lax.while_loop(
            out_cond, out_body,
            (jnp.int32(0), jnp.int32(_BIG), jnp.bool_(False)))
        return fkey, bail

    def full_logic(buf, opp_min):
        has_opp = opp_min < _BIG
        opp_flat = jnp.where(has_opp, opp_min, 0)
        fkey, _ = phase2_scan(buf, opp_min, windowed=False)
        tgt = jnp.where(fkey < _BIG, fkey & (_HW - 1), 0)
        opp_is_start = has_opp & (opp_flat == 3 * _W + 6)
        common = has_opp & (~opp_is_start)

        def count_food(_):
            def body(r, cnt):
                for j in range(_GPR):
                    m = buf[r, pl.ds(16 * j, _L)] == 1.0
                    cnt = cnt + m.astype(jnp.int32)
                return cnt

            return jnp.sum(lax.fori_loop(0, _H, body,
                                         jnp.zeros((_L,), jnp.int32)))

        n_food = lax.cond(common, lambda _: jnp.int32(2), count_food, 0)

        has_food = fkey < _BIG
        use_argmin = jnp.where(common, has_food, n_food == 1)
        target = jnp.where(use_argmin, tgt, 0)
        val = jnp.where(jnp.where(common, has_food, n_food > 0),
                        jnp.float32(1.0), jnp.float32(0.0))
        return target, val

    def sample(b, xb3, xb1, c3, c1, s3, s1):
        c3.wait()
        mnw = phase1(xb3, _W3 // 8)
        c1.wait()
        fkey_w, bail = phase2_scan(xb1, mnw, windowed=True)
        has_opp_w = mnw < _BIG
        opp_is_start = has_opp_w & (mnw == 3 * _W + 6)
        fast_ok = has_opp_w & (~opp_is_start) & (~bail)

        def fast(_):
            has_food = fkey_w < _BIG
            return (jnp.where(has_food, fkey_w & (_HW - 1), 0),
                    jnp.where(has_food, jnp.float32(1.0), jnp.float32(0.0)))

        def slow(_):
            base3 = b * (_H * _C) + 3
            for g in range(_H // _L):
                ifull[pl.ds(_L * g, _L)] = base3 + 4 * (iota + _L * g)
            pltpu.async_copy(x_hbm.at[ifull], xb3, s3).wait()
            base1 = b * (_H * _C) + 1
            for g in range(_H // _L):
                ifull[pl.ds(_L * g, _L)] = base1 + 4 * (iota + _L * g)
            pltpu.async_copy(x_hbm.at[ifull], xb1, s1).wait()
            opp = phase1(xb3, _H // 8)
            return full_logic(xb1, opp)

        return lax.cond(fast_ok, fast, slow, 0)

    def scatter(target, val):
        tr_v = jnp.broadcast_to(target >> 7, (_L,))
        tc_v = jnp.broadcast_to(target & (_W - 1), (_L,))
        plsc.store_scatter(zbuf, [tr_v, tc_v],
                           jnp.broadcast_to(val, (_L,)), mask=lane0)
        return tr_v, tc_v

    target0, val0 = sample(b0, xb3a, xb1a, c3a, c1a, s3a, s1a)
    tr0, tc0 = scatter(target0, val0)
    o1a = pltpu.async_copy(zbuf, out1.at[b0], so1)

    target1, val1 = sample(b1, xb3b, xb1b, c3b, c1b, s3b, s1b)
    o1a.wait()
    plsc.store_scatter(zbuf, [tr0, tc0], zeros_v, mask=lane0)
    scatter(target1, val1)
    pltpu.sync_copy(zbuf, out1.at[b1])


_oracle = pl.kernel(
    _oracle_body,
    out_type=jax.ShapeDtypeStruct((_B, _H, _W), jnp.float32),
    mesh=_mesh,
    scratch_types=[pltpu.VMEM((_H, _W), jnp.float32),
                   pltpu.VMEM((_H, _W), jnp.float32),
                   pltpu.VMEM((_H, _W), jnp.float32),
                   pltpu.VMEM((_H, _W), jnp.float32),
                   pltpu.VMEM((_H, _W), jnp.float32),
                   pltpu.VMEM((_W3,), jnp.int32),
                   pltpu.VMEM((_W1,), jnp.int32),
                   pltpu.VMEM((_W3,), jnp.int32),
                   pltpu.VMEM((_W1,), jnp.int32),
                   pltpu.VMEM((_H,), jnp.int32),
                   pltpu.SemaphoreType.DMA,
                   pltpu.SemaphoreType.DMA,
                   pltpu.SemaphoreType.DMA,
                   pltpu.SemaphoreType.DMA,
                   pltpu.SemaphoreType.DMA],
    compiler_params=pltpu.CompilerParams(needs_layout_passes=False),
)


@jax.jit
def kernel(x, history):
    del history
    x_t = jnp.transpose(x, (0, 1, 3, 2)).reshape(_B * _H * _C, _W)
    out1 = _oracle(x_t)
    return out1, jnp.zeros((_B, _H, _W), jnp.float32)
